# Initial kernel scaffold; baseline (speedup 1.0000x reference)
import jax
import jax.numpy as jnp
from jax.experimental import pallas as pl


def kernel(x, pos_table):
    raise NotImplementedError("write your pallas kernel here")


# speedup vs baseline: 1.4807x; 1.4807x over previous
import jax
import jax.numpy as jnp
from jax.experimental import pallas as pl


def _add_body(x_ref, pos_ref, out_ref):
    out_ref[...] = x_ref[...] + pos_ref[...]


def kernel(x, pos_table):
    B, S, D = x.shape
    BS = 256
    nsb = S // BS
    xr = x.reshape(B * S, D)
    out = pl.pallas_call(
        _add_body,
        grid=(nsb, B),
        in_specs=[
            pl.BlockSpec((BS, D), lambda i, b: (b * nsb + i, 0)),
            pl.BlockSpec((BS, D), lambda i, b: (i, 0)),
        ],
        out_specs=pl.BlockSpec((BS, D), lambda i, b: (b * nsb + i, 0)),
        out_shape=jax.ShapeDtypeStruct((B * S, D), x.dtype),
    )(xr, pos_table)
    return out.reshape(B, S, D)
